# Initial kernel scaffold; baseline (speedup 1.0000x reference)
import jax
import jax.numpy as jnp
from jax.experimental import pallas as pl


def kernel(lut, start_emb, pad_emb, state_embs, state_out_gates, state_out_biases, h0_lin, wih, whh, b_ih, b_hh, dec_w, dec_b, trans_weights, trans_bias, init_trans, len_scores, inps, combotargs):
    raise NotImplementedError("write your pallas kernel here")


# speedup vs baseline: 1.4083x; 1.4083x over previous
import jax
import jax.numpy as jnp
from jax.experimental import pallas as pl
from jax.experimental.pallas import tpu as pltpu

NEG = -1e30


def _lstm_decode_body(seg_ref, h0c0_ref, wihx_ref, whh_ref, bk_ref,
                      gate_ref, bias_ref, wdec_ref, bdec_ref,
                      out_ref, h_scr, c_scr):
    t = pl.program_id(1)
    KB, _ = bk_ref.shape
    R = seg_ref.shape[1]
    BB, H = h_scr.shape

    @pl.when(t == 0)
    def _():
        h_scr[...] = jnp.broadcast_to(h0c0_ref[:, 0:H], (BB, H))
        c_scr[...] = jnp.broadcast_to(h0c0_ref[:, H:2 * H], (BB, H))

    x = seg_ref[0]
    xg = jnp.dot(x, wihx_ref[...], preferred_element_type=jnp.float32)
    h = h_scr[...]
    hg = jnp.dot(h, whh_ref[...], preferred_element_type=jnp.float32)

    xgb = jnp.broadcast_to(xg[None], (KB, R, 4 * H)).reshape(BB, 4 * H)
    bkb = jnp.broadcast_to(bk_ref[...][:, None, :], (KB, R, 4 * H)).reshape(BB, 4 * H)
    gates = hg + xgb + bkb

    i = jax.nn.sigmoid(gates[:, 0:H])
    f = jax.nn.sigmoid(gates[:, H:2 * H])
    g = jnp.tanh(gates[:, 2 * H:3 * H])
    o = jax.nn.sigmoid(gates[:, 3 * H:4 * H])
    c_new = f * c_scr[...] + i * g
    h_new = o * jnp.tanh(c_new)
    h_scr[...] = h_new
    c_scr[...] = c_new

    gmul = jnp.broadcast_to(gate_ref[...][:, None, :], (KB, R, H)).reshape(BB, H)
    badd = jnp.broadcast_to(bias_ref[...][:, None, :], (KB, R, H)).reshape(BB, H)
    s = gmul * h_new + badd
    logits = jnp.dot(s, wdec_ref[...], preferred_element_type=jnp.float32) + bdec_ref[...]
    mx = jnp.max(logits, axis=-1, keepdims=True)
    lse = jnp.log(jnp.sum(jnp.exp(logits - mx), axis=-1, keepdims=True)) + mx
    out_ref[0] = (logits - lse).reshape(KB, R, out_ref.shape[-1])


def _lstm_decode(segembs, h0c0, wihx_t, whh_t, bk, gates_k, biases_k,
                 wdec_pad, bdec_pad, nb):
    Lp1, R, E = segembs.shape
    K, H4 = bk.shape
    H = H4 // 4
    V = wdec_pad.shape[1]
    KB = K // nb
    return pl.pallas_call(
        _lstm_decode_body,
        out_shape=jax.ShapeDtypeStruct((Lp1, K, R, V), jnp.float32),
        grid_spec=pltpu.PrefetchScalarGridSpec(
            num_scalar_prefetch=0,
            grid=(nb, Lp1),
            in_specs=[
                pl.BlockSpec((1, R, E), lambda n, t: (t, 0, 0)),
                pl.BlockSpec((1, 2 * H), lambda n, t: (0, 0)),
                pl.BlockSpec((E, H4), lambda n, t: (0, 0)),
                pl.BlockSpec((H, H4), lambda n, t: (0, 0)),
                pl.BlockSpec((KB, H4), lambda n, t: (n, 0)),
                pl.BlockSpec((KB, H), lambda n, t: (n, 0)),
                pl.BlockSpec((KB, H), lambda n, t: (n, 0)),
                pl.BlockSpec((H, V), lambda n, t: (0, 0)),
                pl.BlockSpec((1, V), lambda n, t: (0, 0)),
            ],
            out_specs=pl.BlockSpec((1, KB, R, V), lambda n, t: (t, n, 0, 0)),
            scratch_shapes=[pltpu.VMEM((KB * R, H), jnp.float32),
                            pltpu.VMEM((KB * R, H), jnp.float32)],
        ),
        compiler_params=pltpu.CompilerParams(
            dimension_semantics=("parallel", "arbitrary")),
    )(segembs, h0c0, wihx_t, whh_t, bk, gates_k, biases_k, wdec_pad, bdec_pad)


def kernel(lut, start_emb, pad_emb, state_embs, state_out_gates, state_out_biases,
           h0_lin, wih, whh, b_ih, b_hh, dec_w, dec_b, trans_weights, trans_bias,
           init_trans, len_scores, inps, combotargs):
    K = state_embs.shape[0]
    L = len_scores.shape[1]
    E = start_emb.shape[-1]
    H = whh.shape[1]
    hsmm_emb = state_embs.shape[-1]
    gentypes = dec_w.shape[0] - 1
    eop_idx = gentypes
    VPAD = 128
    bsz, seqlen = inps.shape
    NB = 2

    state_embs2d = state_embs.reshape(K, hsmm_emb)
    selfmask = jnp.diag(jnp.full((K,), NEG, jnp.float32))
    sc = (state_embs2d @ trans_weights @ state_embs2d.T + trans_bias + selfmask)
    tscores = jax.nn.log_softmax(sc, axis=1)
    trans_lps = jnp.broadcast_to(tscores[None, None], (seqlen - 1, bsz, K, K))
    init_lps = jnp.broadcast_to(jax.nn.log_softmax(init_trans, axis=1), (bsz, K))

    len_sc = jnp.broadcast_to(len_scores, (K, L))
    lplist = [jnp.zeros((1, K), jnp.float32)]
    for l in range(2, L + 1):
        lplist.append(jax.nn.log_softmax(len_sc[:, :l], axis=1).T)

    embs = jnp.take(lut, inps.T, axis=0)
    xemb = jnp.transpose(embs, (1, 0, 2))
    rows = [jnp.broadcast_to(start_emb, (bsz, seqlen, E)), xemb]
    for i in range(1, L):
        pad = jnp.broadcast_to(pad_emb, (bsz, i, E))
        rows.append(jnp.concatenate([xemb[:, i:], pad], axis=1))
    segembs = jnp.stack(rows).reshape(L + 1, bsz * seqlen, E)
    Lp1, bszsl, _ = segembs.shape

    h0c0 = jnp.concatenate([jnp.tanh(h0_lin[:H]), h0_lin[H:]]).reshape(1, 2 * H)
    wih_t = wih.T
    wihx_t = wih_t[:E]
    wihs_t = wih_t[E:]
    b = (b_ih + b_hh).reshape(1, 4 * H)
    bk = b + state_embs2d @ wihs_t
    gates_k = state_out_gates.reshape(K, H)
    biases_k = state_out_biases.reshape(K, H)

    wdec_t = dec_w.T
    wdec_pad = jnp.zeros((H, VPAD), jnp.float32).at[:, :gentypes + 1].set(wdec_t)
    bdec_pad = jnp.full((1, VPAD), NEG, jnp.float32).at[:, :gentypes + 1].set(
        dec_b.reshape(1, gentypes + 1))

    wlogps = _lstm_decode(segembs, h0c0, wihx_t, whh.T,
                          bk, gates_k, biases_k, wdec_pad, bdec_pad, NB)
    wlogps = wlogps.reshape(Lp1, K, bszsl, VPAD)

    ct = jnp.transpose(combotargs, (1, 0, 2)).reshape(L, bszsl)
    idx = jnp.broadcast_to(ct[:, None, :, None], (L, K, bszsl, 1))
    lls = jnp.take_along_axis(wlogps[:L], idx, axis=-1)[..., 0]
    seglls = jnp.cumsum(lls, axis=0)
    eop_lls = wlogps[1:L + 1, :, :, eop_idx]
    seglls = seglls + eop_lls

    obslps = jnp.transpose(seglls, (1, 0, 2)).reshape(K, L, bsz, seqlen)
    obslps = jnp.transpose(obslps, (1, 3, 2, 0))

    T = seqlen
    beta = [None] * (T + 1)
    beta_star = [None] * T
    beta[T] = jnp.zeros((bsz, K), jnp.float32)
    for t in range(T - 1, -1, -1):
        steps = min(L, T - t)
        len_terms = lplist[steps - 1]
        betas = jnp.stack([beta[t + l + 1] for l in range(steps)])
        obs = obslps[:steps, t]
        terms = betas + obs + len_terms[:, None, :]
        beta_star[t] = jax.nn.logsumexp(terms, axis=0)
        if t > 0:
            bt = beta_star[t][:, None, :] + trans_lps[t - 1]
            beta[t] = jax.nn.logsumexp(bt, axis=2)

    log_marg = jax.nn.logsumexp(beta_star[0] + init_lps, axis=1).sum()
    return log_marg


# speedup vs baseline: 3.1607x; 2.2443x over previous
import jax
import jax.numpy as jnp
from jax.experimental import pallas as pl
from jax.experimental.pallas import tpu as pltpu

NEG = -1e30


def _lstm_decode_body(seg_ref, ohT_ref, h0c0_ref, wihx_ref, whh_ref, bk_ref,
                      gate_ref, bias_ref, wdec_ref, bdec_ref,
                      lls_ref, eop_ref, h_scr, c_scr, *, eop_idx):
    t = pl.program_id(1)
    KB, _ = bk_ref.shape
    R = seg_ref.shape[1]
    BB, H = h_scr.shape

    @pl.when(t == 0)
    def _():
        h_scr[...] = jnp.broadcast_to(h0c0_ref[:, 0:H], (BB, H))
        c_scr[...] = jnp.broadcast_to(h0c0_ref[:, H:2 * H], (BB, H))

    x = seg_ref[0]
    xg = jnp.dot(x, wihx_ref[...], preferred_element_type=jnp.float32)
    h = h_scr[...]
    hg = jnp.dot(h, whh_ref[...], preferred_element_type=jnp.float32)

    xgb = jnp.broadcast_to(xg[None], (KB, R, 4 * H)).reshape(BB, 4 * H)
    bkb = jnp.broadcast_to(bk_ref[...][:, None, :], (KB, R, 4 * H)).reshape(BB, 4 * H)
    gates = hg + xgb + bkb

    i = jax.nn.sigmoid(gates[:, 0:H])
    f = jax.nn.sigmoid(gates[:, H:2 * H])
    g = jnp.tanh(gates[:, 2 * H:3 * H])
    o = jax.nn.sigmoid(gates[:, 3 * H:4 * H])
    c_new = f * c_scr[...] + i * g
    h_new = o * jnp.tanh(c_new)
    h_scr[...] = h_new
    c_scr[...] = c_new

    gmul = jnp.broadcast_to(gate_ref[...][:, None, :], (KB, R, H)).reshape(BB, H)
    badd = jnp.broadcast_to(bias_ref[...][:, None, :], (KB, R, H)).reshape(BB, H)
    s = gmul * h_new + badd
    logits = jnp.dot(s, wdec_ref[...], preferred_element_type=jnp.float32) + bdec_ref[...]

    ohT = ohT_ref[0]
    lls_rows, eop_rows = [], []
    for k in range(KB):
        tk = jnp.transpose(logits[k * R:(k + 1) * R, :])
        mx = jnp.max(tk, axis=0, keepdims=True)
        lse = jnp.log(jnp.sum(jnp.exp(tk - mx), axis=0, keepdims=True)) + mx
        lls_rows.append(jnp.sum(tk * ohT, axis=0, keepdims=True) - lse)
        eop_rows.append(tk[eop_idx:eop_idx + 1, :] - lse)
    lls_ref[0] = jnp.concatenate(lls_rows, axis=0)
    eop_ref[0] = jnp.concatenate(eop_rows, axis=0)


def _lstm_decode(segembs, ohT, h0c0, wihx_t, whh_t, bk, gates_k, biases_k,
                 wdec_pad, bdec_pad, nb, eop_idx):
    import functools
    Lp1, R, E = segembs.shape
    K, H4 = bk.shape
    H = H4 // 4
    V = wdec_pad.shape[1]
    KB = K // nb
    return pl.pallas_call(
        functools.partial(_lstm_decode_body, eop_idx=eop_idx),
        out_shape=(jax.ShapeDtypeStruct((Lp1, K, R), jnp.float32),
                   jax.ShapeDtypeStruct((Lp1, K, R), jnp.float32)),
        grid_spec=pltpu.PrefetchScalarGridSpec(
            num_scalar_prefetch=0,
            grid=(nb, Lp1),
            in_specs=[
                pl.BlockSpec((1, R, E), lambda n, t: (t, 0, 0)),
                pl.BlockSpec((1, V, R), lambda n, t: (t, 0, 0)),
                pl.BlockSpec((1, 2 * H), lambda n, t: (0, 0)),
                pl.BlockSpec((E, H4), lambda n, t: (0, 0)),
                pl.BlockSpec((H, H4), lambda n, t: (0, 0)),
                pl.BlockSpec((KB, H4), lambda n, t: (n, 0)),
                pl.BlockSpec((KB, H), lambda n, t: (n, 0)),
                pl.BlockSpec((KB, H), lambda n, t: (n, 0)),
                pl.BlockSpec((H, V), lambda n, t: (0, 0)),
                pl.BlockSpec((1, V), lambda n, t: (0, 0)),
            ],
            out_specs=(pl.BlockSpec((1, KB, R), lambda n, t: (t, n, 0)),
                       pl.BlockSpec((1, KB, R), lambda n, t: (t, n, 0))),
            scratch_shapes=[pltpu.VMEM((KB * R, H), jnp.float32),
                            pltpu.VMEM((KB * R, H), jnp.float32)],
        ),
        compiler_params=pltpu.CompilerParams(
            dimension_semantics=("parallel", "arbitrary")),
    )(segembs, ohT, h0c0, wihx_t, whh_t, bk, gates_k, biases_k,
      wdec_pad, bdec_pad)


def _dp_body(lls_ref, eop_ref, se_ref, tw_ref, tb_ref, lsc_ref, initT_ref,
             out_ref, *, L, bsz, seqlen):
    K = se_ref.shape[0]
    T = seqlen

    a = jnp.dot(se_ref[...], tw_ref[...], preferred_element_type=jnp.float32)
    sc = jax.lax.dot_general(a, se_ref[...], (((1,), (1,)), ((), ())),
                             preferred_element_type=jnp.float32)
    ii = jax.lax.broadcasted_iota(jnp.int32, (K, K), 0)
    jj = jax.lax.broadcasted_iota(jnp.int32, (K, K), 1)
    sc = sc + tb_ref[...] + jnp.where(ii == jj, NEG, 0.0)
    mx = jnp.max(sc, axis=1, keepdims=True)
    tsc = sc - mx - jnp.log(jnp.sum(jnp.exp(sc - mx), axis=1, keepdims=True))
    expT = jnp.exp(tsc)

    lsc = lsc_ref[...]
    len_scal = {}
    for s in range(L):
        v = lsc[:, :s + 1]
        m = jnp.max(v, axis=1, keepdims=True)
        ls = v - m - jnp.log(jnp.sum(jnp.exp(v - m), axis=1, keepdims=True))
        for l in range(s + 1):
            len_scal[(s, l)] = ls[0, l]

    it = initT_ref[...]
    m0 = jnp.max(it, axis=0, keepdims=True)
    init_col = it - m0 - jnp.log(jnp.sum(jnp.exp(it - m0), axis=0, keepdims=True))

    cum = lls_ref[0]
    obs = []
    for l in range(L):
        if l > 0:
            cum = cum + lls_ref[l]
        obs.append(cum + eop_ref[l + 1])

    zeros = jnp.zeros((K, bsz), jnp.float32)
    beta = {T: zeros}
    bs0 = None
    for t in range(T - 1, -1, -1):
        steps = min(L, T - t)
        terms = []
        for l in range(steps):
            b_next = beta.get(t + l + 1, zeros)
            o = obs[l][:, t * bsz:(t + 1) * bsz]
            terms.append(b_next + o + len_scal[(steps - 1, l)])
        if steps == 1:
            bs = terms[0]
        else:
            m = terms[0]
            for tm in terms[1:]:
                m = jnp.maximum(m, tm)
            acc = jnp.exp(terms[0] - m)
            for tm in terms[1:]:
                acc = acc + jnp.exp(tm - m)
            bs = jnp.log(acc) + m
        bs0 = bs
        if t > 0:
            m2 = jnp.max(bs, axis=0, keepdims=True)
            p = jnp.exp(bs - m2)
            beta[t] = jnp.log(
                jnp.dot(expT, p, preferred_element_type=jnp.float32)) + m2

    fin = bs0 + init_col
    mf = jnp.max(fin, axis=0, keepdims=True)
    lsef = jnp.log(jnp.sum(jnp.exp(fin - mf), axis=0, keepdims=True)) + mf
    out_ref[...] = jnp.sum(lsef, axis=1, keepdims=True)


def _dp_call(lls, eop, se2d, trans_weights, trans_bias, len_scores, initT,
             L, bsz, seqlen):
    import functools
    K = se2d.shape[0]
    body = functools.partial(_dp_body, L=L, bsz=bsz, seqlen=seqlen)
    return pl.pallas_call(
        body,
        out_shape=jax.ShapeDtypeStruct((1, 1), jnp.float32),
        in_specs=[pl.BlockSpec(lls.shape, lambda i: (0, 0, 0)),
                  pl.BlockSpec(eop.shape, lambda i: (0, 0, 0)),
                  pl.BlockSpec(se2d.shape, lambda i: (0, 0)),
                  pl.BlockSpec(trans_weights.shape, lambda i: (0, 0)),
                  pl.BlockSpec(trans_bias.shape, lambda i: (0, 0)),
                  pl.BlockSpec(len_scores.shape, lambda i: (0, 0)),
                  pl.BlockSpec(initT.shape, lambda i: (0, 0))],
        out_specs=pl.BlockSpec((1, 1), lambda i: (0, 0)),
        compiler_params=pltpu.CompilerParams(
            dimension_semantics=("arbitrary",)),
        grid=(1,),
    )(lls, eop, se2d, trans_weights, trans_bias, len_scores, initT)


def kernel(lut, start_emb, pad_emb, state_embs, state_out_gates, state_out_biases,
           h0_lin, wih, whh, b_ih, b_hh, dec_w, dec_b, trans_weights, trans_bias,
           init_trans, len_scores, inps, combotargs):
    K = state_embs.shape[0]
    L = len_scores.shape[1]
    E = start_emb.shape[-1]
    H = whh.shape[1]
    hsmm_emb = state_embs.shape[-1]
    gentypes = dec_w.shape[0] - 1
    VPAD = 128
    bsz, seqlen = inps.shape
    NB = 2

    embs = jnp.take(lut, inps.T, axis=0)
    rows = [jnp.broadcast_to(start_emb.reshape(1, 1, E), (seqlen, bsz, E)), embs]
    for i in range(1, L):
        pad = jnp.broadcast_to(pad_emb.reshape(1, 1, E), (i, bsz, E))
        rows.append(jnp.concatenate([embs[i:], pad], axis=0))
    segembs = jnp.stack(rows).reshape(L + 1, seqlen * bsz, E)

    ct = jnp.transpose(combotargs, (1, 2, 0)).reshape(L, seqlen * bsz)
    vio = jnp.arange(VPAD, dtype=ct.dtype)[None, :, None]
    ohT = (vio == ct[:, None, :]).astype(jnp.float32)
    ohT = jnp.concatenate([ohT, jnp.zeros((1, VPAD, seqlen * bsz), jnp.float32)], 0)

    h0c0 = jnp.concatenate([jnp.tanh(h0_lin[:H]), h0_lin[H:]]).reshape(1, 2 * H)
    wih_t = wih.T
    wihx_t = wih_t[:E]
    state_embs2d = state_embs.reshape(K, hsmm_emb)
    b = (b_ih + b_hh).reshape(1, 4 * H)
    bk = b + state_embs2d @ wih_t[E:]
    gates_k = state_out_gates.reshape(K, H)
    biases_k = state_out_biases.reshape(K, H)

    wdec_pad = jnp.zeros((H, VPAD), jnp.float32).at[:, :gentypes + 1].set(dec_w.T)
    bdec_pad = jnp.full((1, VPAD), NEG, jnp.float32).at[:, :gentypes + 1].set(
        dec_b.reshape(1, gentypes + 1))

    lls, eop = _lstm_decode(segembs, ohT, h0c0, wihx_t, whh.T, bk,
                            gates_k, biases_k, wdec_pad, bdec_pad, NB, gentypes)

    out = _dp_call(lls, eop, state_embs2d, trans_weights, trans_bias,
                   len_scores, init_trans.T, L, bsz, seqlen)
    return out.reshape(())


# speedup vs baseline: 3.1757x; 1.0047x over previous
import functools

import jax
import jax.numpy as jnp
from jax.experimental import pallas as pl
from jax.experimental.pallas import tpu as pltpu

NEG = -1e30


def _lstm_decode_body(xp_ref, start_ref, ct_ref, h0_ref, wih_ref, whh_ref,
                      se_ref, bih_ref, bhh_ref, gate_ref, bias_ref,
                      wdec_ref, bdec_ref,
                      lls_ref, eop_ref, h_scr, c_scr, bk_scr,
                      *, eop_idx, bsz):
    t = pl.program_id(1)
    KB = gate_ref.shape[0]
    E = start_ref.shape[1]
    BB, H = h_scr.shape
    R = BB // KB
    V = wdec_ref.shape[1]

    @pl.when(t == 0)
    def _():
        h_scr[...] = jnp.broadcast_to(jnp.tanh(h0_ref[:, 0:H]), (BB, H))
        c_scr[...] = jnp.broadcast_to(h0_ref[:, H:2 * H], (BB, H))
        bk_scr[...] = (bih_ref[...] + bhh_ref[...] +
                       jax.lax.dot_general(
                           se_ref[...], wih_ref[:, E:],
                           (((1,), (1,)), ((), ())),
                           preferred_element_type=jnp.float32))

    off = pl.multiple_of(jnp.maximum(t - 1, 0) * bsz, bsz)
    xw = xp_ref[pl.ds(off, R), :]
    x = jnp.where(t == 0, jnp.broadcast_to(start_ref[...], (R, E)), xw)

    xg = jax.lax.dot_general(x, wih_ref[:, 0:E], (((1,), (1,)), ((), ())),
                             preferred_element_type=jnp.float32)
    h = h_scr[...]
    hg = jax.lax.dot_general(h, whh_ref[...], (((1,), (1,)), ((), ())),
                             preferred_element_type=jnp.float32)

    xgb = jnp.broadcast_to(xg[None], (KB, R, 4 * H)).reshape(BB, 4 * H)
    bkb = jnp.broadcast_to(bk_scr[...][:, None, :], (KB, R, 4 * H)).reshape(BB, 4 * H)
    gates = hg + xgb + bkb

    i = jax.nn.sigmoid(gates[:, 0:H])
    f = jax.nn.sigmoid(gates[:, H:2 * H])
    g = jnp.tanh(gates[:, 2 * H:3 * H])
    o = jax.nn.sigmoid(gates[:, 3 * H:4 * H])
    c_new = f * c_scr[...] + i * g
    h_new = o * jnp.tanh(c_new)
    h_scr[...] = h_new
    c_scr[...] = c_new

    gmul = jnp.broadcast_to(gate_ref[...][:, None, :], (KB, R, H)).reshape(BB, H)
    badd = jnp.broadcast_to(bias_ref[...][:, None, :], (KB, R, H)).reshape(BB, H)
    s = gmul * h_new + badd
    logits = jnp.dot(s, wdec_ref[...], preferred_element_type=jnp.float32) + bdec_ref[...]

    ctrow = ct_ref[0]
    mask = (jax.lax.broadcasted_iota(jnp.int32, (V, R), 0)
            == jnp.broadcast_to(ctrow, (V, R)))

    lls_rows, eop_rows = [], []
    for k in range(KB):
        tk = jnp.transpose(logits[k * R:(k + 1) * R, :])
        mx = jnp.max(tk, axis=0, keepdims=True)
        lse = jnp.log(jnp.sum(jnp.exp(tk - mx), axis=0, keepdims=True)) + mx
        lls_rows.append(jnp.sum(jnp.where(mask, tk, 0.0), axis=0, keepdims=True) - lse)
        eop_rows.append(tk[eop_idx:eop_idx + 1, :] - lse)
    lls_ref[0] = jnp.concatenate(lls_rows, axis=0)
    eop_ref[0] = jnp.concatenate(eop_rows, axis=0)


def _lstm_decode(xp, start_row, ct3, h0_row, wih, whh, se2d, bih, bhh,
                 gates_k, biases_k, wdec_pad, bdec_pad, nb, eop_idx, Lp1, bsz):
    K, H = gates_k.shape
    E = start_row.shape[1]
    V = wdec_pad.shape[1]
    L, _, R = ct3.shape
    KB = K // nb
    body = functools.partial(_lstm_decode_body, eop_idx=eop_idx, bsz=bsz)
    return pl.pallas_call(
        body,
        out_shape=(jax.ShapeDtypeStruct((Lp1, K, R), jnp.float32),
                   jax.ShapeDtypeStruct((Lp1, K, R), jnp.float32)),
        grid_spec=pltpu.PrefetchScalarGridSpec(
            num_scalar_prefetch=0,
            grid=(nb, Lp1),
            in_specs=[
                pl.BlockSpec(xp.shape, lambda n, t: (0, 0)),
                pl.BlockSpec((1, E), lambda n, t: (0, 0)),
                pl.BlockSpec((1, 1, R), lambda n, t: (jnp.minimum(t, L - 1), 0, 0)),
                pl.BlockSpec((1, 2 * H), lambda n, t: (0, 0)),
                pl.BlockSpec(wih.shape, lambda n, t: (0, 0)),
                pl.BlockSpec(whh.shape, lambda n, t: (0, 0)),
                pl.BlockSpec((KB, se2d.shape[1]), lambda n, t: (n, 0)),
                pl.BlockSpec((1, 4 * H), lambda n, t: (0, 0)),
                pl.BlockSpec((1, 4 * H), lambda n, t: (0, 0)),
                pl.BlockSpec((KB, H), lambda n, t: (n, 0)),
                pl.BlockSpec((KB, H), lambda n, t: (n, 0)),
                pl.BlockSpec((H, V), lambda n, t: (0, 0)),
                pl.BlockSpec((1, V), lambda n, t: (0, 0)),
            ],
            out_specs=(pl.BlockSpec((1, KB, R), lambda n, t: (t, n, 0)),
                       pl.BlockSpec((1, KB, R), lambda n, t: (t, n, 0))),
            scratch_shapes=[pltpu.VMEM((KB * R, H), jnp.float32),
                            pltpu.VMEM((KB * R, H), jnp.float32),
                            pltpu.VMEM((KB, 4 * H), jnp.float32)],
        ),
        compiler_params=pltpu.CompilerParams(
            dimension_semantics=("parallel", "arbitrary")),
    )(xp, start_row, ct3, h0_row, wih, whh, se2d, bih, bhh,
      gates_k, biases_k, wdec_pad, bdec_pad)


def _dp_body(lls_ref, eop_ref, se_ref, tw_ref, tb_ref, lsc_ref, init_ref,
             out_ref, *, L, bsz, seqlen):
    K = se_ref.shape[0]
    T = seqlen

    a = jnp.dot(se_ref[...], tw_ref[...], preferred_element_type=jnp.float32)
    sc = jax.lax.dot_general(a, se_ref[...], (((1,), (1,)), ((), ())),
                             preferred_element_type=jnp.float32)
    ii = jax.lax.broadcasted_iota(jnp.int32, (K, K), 0)
    jj = jax.lax.broadcasted_iota(jnp.int32, (K, K), 1)
    sc = sc + tb_ref[...] + jnp.where(ii == jj, NEG, 0.0)
    mx = jnp.max(sc, axis=1, keepdims=True)
    tsc = sc - mx - jnp.log(jnp.sum(jnp.exp(sc - mx), axis=1, keepdims=True))
    expT = jnp.exp(tsc)

    lsc = lsc_ref[...]
    len_scal = {}
    for s in range(L):
        v = lsc[:, :s + 1]
        m = jnp.max(v, axis=1, keepdims=True)
        ls = v - m - jnp.log(jnp.sum(jnp.exp(v - m), axis=1, keepdims=True))
        for l in range(s + 1):
            len_scal[(s, l)] = ls[0, l]

    vi = init_ref[...]
    mi = jnp.max(vi, axis=1, keepdims=True)
    ils = vi - mi - jnp.log(jnp.sum(jnp.exp(vi - mi), axis=1, keepdims=True))
    pinit = jnp.exp(ils)

    cum = lls_ref[0]
    obs = []
    for l in range(L):
        if l > 0:
            cum = cum + lls_ref[l]
        obs.append(cum + eop_ref[l + 1])

    zeros = jnp.zeros((K, bsz), jnp.float32)
    beta = {T: zeros}
    bs0 = None
    for t in range(T - 1, -1, -1):
        steps = min(L, T - t)
        terms = []
        for l in range(steps):
            b_next = beta.get(t + l + 1, zeros)
            o = obs[l][:, t * bsz:(t + 1) * bsz]
            terms.append(b_next + o + len_scal[(steps - 1, l)])
        if steps == 1:
            bs = terms[0]
        else:
            m = terms[0]
            for tm in terms[1:]:
                m = jnp.maximum(m, tm)
            acc = jnp.exp(terms[0] - m)
            for tm in terms[1:]:
                acc = acc + jnp.exp(tm - m)
            bs = jnp.log(acc) + m
        bs0 = bs
        if t > 0:
            m2 = jnp.max(bs, axis=0, keepdims=True)
            p = jnp.exp(bs - m2)
            beta[t] = jnp.log(
                jnp.dot(expT, p, preferred_element_type=jnp.float32)) + m2

    mf = jnp.max(bs0, axis=0, keepdims=True)
    fin = jnp.log(jnp.dot(pinit, jnp.exp(bs0 - mf),
                          preferred_element_type=jnp.float32)) + mf
    out_ref[...] = jnp.sum(fin, axis=1, keepdims=True)


def _dp_call(lls, eop, se2d, trans_weights, trans_bias, len_scores, init_trans,
             L, bsz, seqlen):
    body = functools.partial(_dp_body, L=L, bsz=bsz, seqlen=seqlen)
    return pl.pallas_call(
        body,
        out_shape=jax.ShapeDtypeStruct((1, 1), jnp.float32),
        in_specs=[pl.BlockSpec(lls.shape, lambda i: (0, 0, 0)),
                  pl.BlockSpec(eop.shape, lambda i: (0, 0, 0)),
                  pl.BlockSpec(se2d.shape, lambda i: (0, 0)),
                  pl.BlockSpec(trans_weights.shape, lambda i: (0, 0)),
                  pl.BlockSpec(trans_bias.shape, lambda i: (0, 0)),
                  pl.BlockSpec(len_scores.shape, lambda i: (0, 0)),
                  pl.BlockSpec(init_trans.shape, lambda i: (0, 0))],
        out_specs=pl.BlockSpec((1, 1), lambda i: (0, 0)),
        compiler_params=pltpu.CompilerParams(
            dimension_semantics=("arbitrary",)),
        grid=(1,),
    )(lls, eop, se2d, trans_weights, trans_bias, len_scores, init_trans)


def kernel(lut, start_emb, pad_emb, state_embs, state_out_gates, state_out_biases,
           h0_lin, wih, whh, b_ih, b_hh, dec_w, dec_b, trans_weights, trans_bias,
           init_trans, len_scores, inps, combotargs):
    K = state_embs.shape[0]
    L = len_scores.shape[1]
    E = start_emb.shape[-1]
    H = whh.shape[1]
    hsmm_emb = state_embs.shape[-1]
    gentypes = dec_w.shape[0] - 1
    VPAD = 128
    bsz, seqlen = inps.shape
    NB = 2

    embs = jnp.take(lut, inps.T, axis=0)
    xp = jnp.concatenate(
        [embs, jnp.broadcast_to(pad_emb.reshape(1, 1, E), (L - 1, bsz, E))],
        axis=0).reshape((seqlen + L - 1) * bsz, E)

    ct3 = jnp.transpose(combotargs, (1, 2, 0)).reshape(L, 1, seqlen * bsz)

    wdec_pad = jnp.zeros((H, VPAD), jnp.float32).at[:, :gentypes + 1].set(dec_w.T)
    bdec_pad = jnp.full((1, VPAD), NEG, jnp.float32).at[:, :gentypes + 1].set(
        dec_b.reshape(1, gentypes + 1))

    se2d = state_embs.reshape(K, hsmm_emb)
    lls, eop = _lstm_decode(
        xp, start_emb.reshape(1, E), ct3, h0_lin.reshape(1, 2 * H),
        wih, whh, se2d, b_ih.reshape(1, 4 * H), b_hh.reshape(1, 4 * H),
        state_out_gates.reshape(K, H), state_out_biases.reshape(K, H),
        wdec_pad, bdec_pad, NB, gentypes, L + 1, bsz)

    out = _dp_call(lls, eop, se2d, trans_weights, trans_bias,
                   len_scores, init_trans, L, bsz, seqlen)
    return out.reshape(())
